# baseline (speedup 1.0000x reference)
import functools

import jax
import jax.numpy as jnp
from jax import lax
from jax.experimental import pallas as pl
from jax.experimental.pallas import tpu as pltpu
from jax.experimental.pallas import tpu_sc as plsc

NC = 2
NS = 16
NW = NC * NS
G = 128
H = 50
D = 64
NB = 16384 // G * H
BPW = NB // NW
RING = 4


@jax.jit
def _sc_gather(table, idx_g):
    mesh = plsc.VectorSubcoreMesh(core_axis_name="c", subcore_axis_name="s")

    @functools.partial(
        pl.kernel,
        out_type=jax.ShapeDtypeStruct((H, 8, G, 1024), jnp.float32),
        mesh=mesh,
        scratch_types=(
            [pltpu.VMEM((BPW, G), jnp.int32)]
            + [pltpu.VMEM((G, D), jnp.float32)] * RING
            + [pltpu.VMEM((8, G * 8), jnp.float32)] * RING
            + [pltpu.SemaphoreType.DMA] * (2 * RING)
        ),
        compiler_params=pltpu.CompilerParams(use_tc_tiling_on_sc=False,
                                             needs_layout_passes=False),
    )
    def k(table_hbm, idx_hbm, out_hbm, idx_v, *bufs):
        gbuf = bufs[:RING]
        tbuf = bufs[RING:2 * RING]
        gsem = bufs[2 * RING:3 * RING]
        ssem = bufs[3 * RING:4 * RING]

        wid = lax.axis_index("s") * NC + lax.axis_index("c")
        t_base = wid * BPW

        pltpu.sync_copy(idx_hbm.at[pl.ds(t_base, BPW)], idx_v)

        lanes = lax.iota(jnp.int32, 16)
        hi_l = lanes >> 3
        lo_lG = (lanes & 7) * G

        def fire_gather(n, p):
            pltpu.async_copy(table_hbm.at[idx_v.at[n]], gbuf[p], gsem[p])

        def wait_gather(p):
            pltpu.make_async_copy(table_hbm.at[pl.ds(0, G)],
                                  gbuf[p], gsem[p]).wait()

        def transpose(p):
            @plsc.parallel_loop(0, G, unroll=4)
            def _(r):
                for cq in range(D // 16):
                    v = gbuf[p][r, pl.ds(cq * 16, 16)]
                    plsc.store_scatter(
                        tbuf[p], [hi_l + cq * 2, lo_lG + r], v)

        def fire_store(n, p):
            t = t_base + n
            h = t >> 7
            j = t & (G - 1)
            pltpu.async_copy(tbuf[p], out_hbm.at[h, pl.ds(0, 8), j], ssem[p])

        def wait_store(p):
            pltpu.make_async_copy(tbuf[p], out_hbm.at[0, pl.ds(0, 8), 0],
                                  ssem[p]).wait()

        for p in range(RING):
            fire_gather(p, p)

        def body(i, carry):
            for u in range(RING):
                n = RING * i + u
                wait_gather(u)

                @pl.when(i > 0)
                def _():
                    wait_store(u)

                transpose(u)
                fire_store(n, u)

                @pl.when(i < BPW // RING - 1)
                def _():
                    fire_gather(n + RING, u)

            return carry

        lax.fori_loop(0, BPW // RING, body, None)
        for p in range(RING):
            wait_store(p)

    return k(table, idx_g)


def kernel(input_ids, embedding):
    idx_g = input_ids.astype(jnp.int32).T.reshape(NB, G)
    out = _sc_gather(embedding, idx_g)
    return (out.reshape(H, 8, G, 8, G)
            .transpose((2, 4, 0, 1, 3))
            .reshape(input_ids.shape[0], H, D))


# speedup vs baseline: 1.3511x; 1.3511x over previous
import functools

import jax
import jax.numpy as jnp
from jax import lax
from jax.experimental import pallas as pl
from jax.experimental.pallas import tpu as pltpu
from jax.experimental.pallas import tpu_sc as plsc

NC = 2
NS = 16
NW = NC * NS
G = 128
H = 50
D = 64
NB = 16384 // G * H
BPW = NB // NW
RING = 4


@jax.jit
def _sc_gather(table, idx_g):
    mesh = plsc.VectorSubcoreMesh(core_axis_name="c", subcore_axis_name="s")

    @functools.partial(
        pl.kernel,
        out_type=jax.ShapeDtypeStruct((H, 8, G, 1024), jnp.float32),
        mesh=mesh,
        scratch_types=(
            [pltpu.VMEM((BPW, G), jnp.int32)]
            + [pltpu.VMEM((G, D), jnp.float32)] * RING
            + [pltpu.VMEM((8, G * 8), jnp.float32)] * RING
            + [pltpu.SemaphoreType.DMA] * (2 * RING)
        ),
        compiler_params=pltpu.CompilerParams(use_tc_tiling_on_sc=False,
                                             needs_layout_passes=False),
    )
    def k(table_hbm, idx_hbm, out_hbm, idx_v, *bufs):
        gbuf = bufs[:RING]
        tbuf = bufs[RING:2 * RING]
        gsem = bufs[2 * RING:3 * RING]
        ssem = bufs[3 * RING:4 * RING]

        wid = lax.axis_index("s") * NC + lax.axis_index("c")
        t_base = wid * BPW

        pltpu.sync_copy(idx_hbm.at[pl.ds(t_base, BPW)], idx_v)

        lanes = lax.iota(jnp.int32, 16)
        perm = [(lanes + q) & 15 for q in range(16)]
        perm_hi = [p >> 3 for p in perm]
        perm_lo = [(p & 7) * G for p in perm]

        def fire_gather(n, p):
            pltpu.async_copy(table_hbm.at[idx_v.at[n]], gbuf[p], gsem[p])

        def wait_gather(p):
            pltpu.make_async_copy(table_hbm.at[pl.ds(0, G)],
                                  gbuf[p], gsem[p]).wait()

        def transpose(p):
            @plsc.parallel_loop(0, G, step=16, unroll=2)
            def _(r0):
                rows = lanes + r0
                for cq in range(D // 16):
                    for q in range(16):
                        v = plsc.load_gather(gbuf[p], [rows, perm[q] + cq * 16])
                        plsc.store_scatter(
                            tbuf[p], [perm_hi[q] + cq * 2, perm_lo[q] + rows],
                            v)

        def fire_store(n, p):
            t = t_base + n
            h = t >> 7
            j = t & (G - 1)
            pltpu.async_copy(tbuf[p], out_hbm.at[h, pl.ds(0, 8), j], ssem[p])

        def wait_store(p):
            pltpu.make_async_copy(tbuf[p], out_hbm.at[0, pl.ds(0, 8), 0],
                                  ssem[p]).wait()

        for p in range(RING):
            fire_gather(p, p)

        def body(i, carry):
            for u in range(RING):
                n = RING * i + u
                wait_gather(u)

                @pl.when(i > 0)
                def _():
                    wait_store(u)

                transpose(u)
                fire_store(n, u)

                @pl.when(i < BPW // RING - 1)
                def _():
                    fire_gather(n + RING, u)

            return carry

        lax.fori_loop(0, BPW // RING, body, None)
        for p in range(RING):
            wait_store(p)

    return k(table, idx_g)


def kernel(input_ids, embedding):
    idx_g = input_ids.astype(jnp.int32).T.reshape(NB, G)
    out = _sc_gather(embedding, idx_g)
    return (out.reshape(H, 8, G, 8, G)
            .transpose((2, 4, 0, 1, 3))
            .reshape(input_ids.shape[0], H, D))


# speedup vs baseline: 1.6311x; 1.2073x over previous
import functools

import jax
import jax.numpy as jnp
from jax import lax
from jax.experimental import pallas as pl
from jax.experimental.pallas import tpu as pltpu
from jax.experimental.pallas import tpu_sc as plsc

NC = 2
NS = 16
NW = NC * NS
G = 128
H = 50
D = 64
NB = 16384 // G * H
BPW = NB // NW
RING = 4


V = 1000000
NJB = V // G


def _sc_repack(table_t, tail_rm):
    mesh = plsc.VectorSubcoreMesh(core_axis_name="c", subcore_axis_name="s")

    @functools.partial(
        pl.kernel,
        out_type=jax.ShapeDtypeStruct((V // 2, G), jnp.float32),
        mesh=mesh,
        scratch_types=(
            [pltpu.VMEM((D, G), jnp.float32)] * 4
            + [pltpu.SemaphoreType.DMA] * 4
        ),
        compiler_params=pltpu.CompilerParams(needs_layout_passes=False),
    )
    def k(tab_hbm, tail_hbm, out_hbm, s0, s1, w0, w1, is0, is1, os0, os1):
        sbuf = (s0, s1)
        wbuf = (w0, w1)
        isem = (is0, is1)
        osem = (os0, os1)

        wid = lax.axis_index("s") * NC + lax.axis_index("c")

        lanes = lax.iota(jnp.int32, 16)
        perm = [(lanes + q) & 15 for q in range(16)]

        def fire_load(kk, p):
            j = wid + kk * NW
            pltpu.async_copy(tab_hbm.at[:, pl.ds(j * G, G)], sbuf[p], isem[p])

        def wait_load(p):
            pltpu.make_async_copy(tab_hbm.at[:, pl.ds(0, G)],
                                  sbuf[p], isem[p]).wait()

        def transpose(p, rgroups):
            @plsc.parallel_loop(0, rgroups * 16, step=16, unroll=2)
            def _(r0):
                rows = lanes + r0
                rhalf = rows >> 1
                rodd = (rows & 1) << 6
                for cq in range(D // 16):
                    for q in range(16):
                        cvec = perm[q] + cq * 16
                        v = plsc.load_gather(sbuf[p], [cvec, rows])
                        plsc.store_scatter(wbuf[p], [rhalf, rodd + cvec], v)

        def fire_store(kk, p):
            j = wid + kk * NW
            pltpu.async_copy(wbuf[p], out_hbm.at[pl.ds(j * (G // 2), D)],
                             osem[p])

        def wait_store(p):
            pltpu.make_async_copy(wbuf[p], out_hbm.at[pl.ds(0, D)],
                                  osem[p]).wait()

        KMIN = NJB // NW
        fire_load(0, 0)
        fire_load(1, 1)

        def body(i, carry):
            for u in range(2):
                kk = 2 * i + u
                wait_load(u)

                @pl.when(i > 0)
                def _():
                    wait_store(u)

                transpose(u, 8)
                fire_store(kk, u)

                nxt = kk + 2

                @pl.when(nxt < KMIN)
                def _():
                    fire_load(nxt, u)

            return carry

        lax.fori_loop(0, KMIN // 2, body, None)

        @pl.when(wid < NJB - KMIN * NW)
        def _():
            fire_load(KMIN, 0)
            wait_load(0)
            wait_store(0)
            transpose(0, 8)
            fire_store(KMIN, 0)
            wait_store(0)

        wait_store(1)

        @pl.when(wid >= NJB - KMIN * NW)
        def _():
            wait_store(0)

        @pl.when(wid == NW - 1)
        def _():
            pltpu.sync_copy(tail_hbm, wbuf[0].at[pl.ds(0, 32)])
            pltpu.sync_copy(wbuf[0].at[pl.ds(0, 32)],
                            out_hbm.at[pl.ds(NJB * (G // 2), 32)])

    return k(table_t, tail_rm)


def _sc_gather(table, idx_g):
    mesh = plsc.VectorSubcoreMesh(core_axis_name="c", subcore_axis_name="s")

    @functools.partial(
        pl.kernel,
        out_type=jax.ShapeDtypeStruct((H, 8, G, 1024), jnp.float32),
        mesh=mesh,
        scratch_types=(
            [pltpu.VMEM((BPW, G), jnp.int32)]
            + [pltpu.VMEM((G, D), jnp.float32)] * RING
            + [pltpu.VMEM((8, G * 8), jnp.float32)] * RING
            + [pltpu.SemaphoreType.DMA] * (2 * RING)
        ),
        compiler_params=pltpu.CompilerParams(use_tc_tiling_on_sc=False,
                                             needs_layout_passes=False),
    )
    def k(table_hbm, idx_hbm, out_hbm, idx_v, *bufs):
        gbuf = bufs[:RING]
        tbuf = bufs[RING:2 * RING]
        gsem = bufs[2 * RING:3 * RING]
        ssem = bufs[3 * RING:4 * RING]

        wid = lax.axis_index("s") * NC + lax.axis_index("c")
        t_base = wid * BPW

        pltpu.sync_copy(idx_hbm.at[pl.ds(t_base, BPW)], idx_v)

        lanes = lax.iota(jnp.int32, 16)
        perm = [(lanes + q) & 15 for q in range(16)]
        perm_hi = [p >> 3 for p in perm]
        perm_lo = [(p & 7) * G for p in perm]

        def fire_gather(n, p):
            pltpu.async_copy(table_hbm.at[idx_v.at[n]], gbuf[p], gsem[p])

        def wait_gather(p):
            pltpu.make_async_copy(table_hbm.at[pl.ds(0, G)],
                                  gbuf[p], gsem[p]).wait()

        def transpose(p):
            @plsc.parallel_loop(0, G, step=16, unroll=2)
            def _(r0):
                rows = lanes + r0
                for cq in range(D // 16):
                    for q in range(16):
                        v = plsc.load_gather(gbuf[p], [rows, perm[q] + cq * 16])
                        plsc.store_scatter(
                            tbuf[p], [perm_hi[q] + cq * 2, perm_lo[q] + rows],
                            v)

        def fire_store(n, p):
            t = t_base + n
            h = t >> 7
            j = t & (G - 1)
            pltpu.async_copy(tbuf[p], out_hbm.at[h, pl.ds(0, 8), j], ssem[p])

        def wait_store(p):
            pltpu.make_async_copy(tbuf[p], out_hbm.at[0, pl.ds(0, 8), 0],
                                  ssem[p]).wait()

        for p in range(RING):
            fire_gather(p, p)

        def body(i, carry):
            for u in range(RING):
                n = RING * i + u
                wait_gather(u)

                @pl.when(i > 0)
                def _():
                    wait_store(u)

                transpose(u)
                fire_store(n, u)

                @pl.when(i < BPW // RING - 1)
                def _():
                    fire_gather(n + RING, u)

            return carry

        lax.fori_loop(0, BPW // RING, body, None)
        for p in range(RING):
            wait_store(p)

    return k(table, idx_g)


def kernel(input_ids, embedding):
    idx_g = input_ids.astype(jnp.int32).T.reshape(NB, G)
    tail_rm = embedding[V - D:].reshape(32, G)
    packed = _sc_repack(embedding.T, tail_rm)
    table_rm = packed.reshape(V, D)
    out = _sc_gather(table_rm, idx_g)
    return (out.reshape(H, 8, G, 8, G)
            .transpose((2, 4, 0, 1, 3))
            .reshape(input_ids.shape[0], H, D))


# speedup vs baseline: 1.9126x; 1.1726x over previous
import functools

import jax
import jax.numpy as jnp
from jax import lax
from jax.experimental import pallas as pl
from jax.experimental.pallas import tpu as pltpu
from jax.experimental.pallas import tpu_sc as plsc

NC = 2
NS = 16
NW = NC * NS
G = 128
H = 50
D = 64
NB = 16384 // G * H
BPW = NB // NW
RING = 4


V = 1000000
NJB = V // G


def _sc_repack(table_t, tail_rm):
    mesh = plsc.VectorSubcoreMesh(core_axis_name="c", subcore_axis_name="s")

    @functools.partial(
        pl.kernel,
        out_type=jax.ShapeDtypeStruct((V // 2, G), jnp.float32),
        mesh=mesh,
        scratch_types=(
            [pltpu.VMEM((D, G), jnp.float32)] * 8
            + [pltpu.SemaphoreType.DMA] * 8
        ),
        compiler_params=pltpu.CompilerParams(needs_layout_passes=False),
    )
    def k(tab_hbm, tail_hbm, out_hbm, *bufs):
        sbuf = bufs[0:4]
        wbuf = bufs[4:8]
        isem = bufs[8:12]
        osem = bufs[12:16]

        wid = lax.axis_index("s") * NC + lax.axis_index("c")

        lanes = lax.iota(jnp.int32, 16)
        perm = [(lanes + q) & 15 for q in range(16)]

        def fire_load(kk, p):
            j = wid + kk * NW
            pltpu.async_copy(tab_hbm.at[:, pl.ds(j * G, G)], sbuf[p], isem[p])

        def wait_load(p):
            pltpu.make_async_copy(tab_hbm.at[:, pl.ds(0, G)],
                                  sbuf[p], isem[p]).wait()

        def transpose(p, rgroups):
            @plsc.parallel_loop(0, rgroups * 16, step=16, unroll=2)
            def _(r0):
                rows = lanes + r0
                rhalf = rows >> 1
                rodd = (rows & 1) << 6
                for cq in range(D // 16):
                    for q in range(16):
                        cvec = perm[q] + cq * 16
                        v = plsc.load_gather(sbuf[p], [cvec, rows])
                        plsc.store_scatter(wbuf[p], [rhalf, rodd + cvec], v)

        def fire_store(kk, p):
            j = wid + kk * NW
            pltpu.async_copy(wbuf[p], out_hbm.at[pl.ds(j * (G // 2), D)],
                             osem[p])

        def wait_store(p):
            pltpu.make_async_copy(wbuf[p], out_hbm.at[pl.ds(0, D)],
                                  osem[p]).wait()

        KMIN = NJB // NW
        for p in range(4):
            fire_load(p, p)

        def body(i, carry):
            for u in range(4):
                kk = 4 * i + u
                wait_load(u)

                @pl.when(i > 0)
                def _():
                    wait_store(u)

                transpose(u, 8)
                fire_store(kk, u)

                nxt = kk + 4

                @pl.when(nxt < KMIN)
                def _():
                    fire_load(nxt, u)

            return carry

        lax.fori_loop(0, KMIN // 4, body, None)

        @pl.when(wid < NJB - KMIN * NW)
        def _():
            fire_load(KMIN, 0)
            wait_load(0)
            wait_store(0)
            transpose(0, 8)
            fire_store(KMIN, 0)
            wait_store(0)

        wait_store(1)
        wait_store(2)
        wait_store(3)

        @pl.when(wid >= NJB - KMIN * NW)
        def _():
            wait_store(0)

        @pl.when(wid == NW - 1)
        def _():
            pltpu.sync_copy(tail_hbm, wbuf[0].at[pl.ds(0, 32)])
            pltpu.sync_copy(wbuf[0].at[pl.ds(0, 32)],
                            out_hbm.at[pl.ds(NJB * (G // 2), 32)])

    return k(table_t, tail_rm)


def _sc_gather(table, idx_g):
    mesh = plsc.VectorSubcoreMesh(core_axis_name="c", subcore_axis_name="s")

    @functools.partial(
        pl.kernel,
        out_type=jax.ShapeDtypeStruct((H, 8, G, 1024), jnp.float32),
        mesh=mesh,
        scratch_types=(
            [pltpu.VMEM((BPW, G), jnp.int32)]
            + [pltpu.VMEM((G, D), jnp.float32)] * RING
            + [pltpu.VMEM((8, G * 8), jnp.float32)] * RING
            + [pltpu.SemaphoreType.DMA] * (2 * RING)
        ),
        compiler_params=pltpu.CompilerParams(use_tc_tiling_on_sc=False,
                                             needs_layout_passes=False),
    )
    def k(table_hbm, idx_hbm, out_hbm, idx_v, *bufs):
        gbuf = bufs[:RING]
        tbuf = bufs[RING:2 * RING]
        gsem = bufs[2 * RING:3 * RING]
        ssem = bufs[3 * RING:4 * RING]

        wid = lax.axis_index("s") * NC + lax.axis_index("c")
        t_base = wid * BPW

        pltpu.sync_copy(idx_hbm.at[pl.ds(t_base, BPW)], idx_v)

        lanes = lax.iota(jnp.int32, 16)
        perm = [(lanes + q) & 15 for q in range(16)]
        perm_hi = [p >> 3 for p in perm]
        perm_lo = [(p & 7) * G for p in perm]

        def fire_gather(n, p):
            pltpu.async_copy(table_hbm.at[idx_v.at[n]], gbuf[p], gsem[p])

        def wait_gather(p):
            pltpu.make_async_copy(table_hbm.at[pl.ds(0, G)],
                                  gbuf[p], gsem[p]).wait()

        def transpose(p):
            @plsc.parallel_loop(0, G, step=16, unroll=4)
            def _(r0):
                rows = lanes + r0
                for cq in range(D // 16):
                    for q in range(16):
                        v = plsc.load_gather(gbuf[p], [rows, perm[q] + cq * 16])
                        plsc.store_scatter(
                            tbuf[p], [perm_hi[q] + cq * 2, perm_lo[q] + rows],
                            v)

        def fire_store(n, p):
            t = t_base + n
            h = t >> 7
            j = t & (G - 1)
            pltpu.async_copy(tbuf[p], out_hbm.at[h, pl.ds(0, 8), j], ssem[p])

        def wait_store(p):
            pltpu.make_async_copy(tbuf[p], out_hbm.at[0, pl.ds(0, 8), 0],
                                  ssem[p]).wait()

        for p in range(RING):
            fire_gather(p, p)

        def body(i, carry):
            for u in range(RING):
                n = RING * i + u
                wait_gather(u)

                @pl.when(i > 0)
                def _():
                    wait_store(u)

                transpose(u)
                fire_store(n, u)

                @pl.when(i < BPW // RING - 1)
                def _():
                    fire_gather(n + RING, u)

            return carry

        lax.fori_loop(0, BPW // RING, body, None)
        for p in range(RING):
            wait_store(p)

    return k(table, idx_g)


def kernel(input_ids, embedding):
    idx_g = input_ids.astype(jnp.int32).T.reshape(NB, G)
    tail_rm = embedding[V - D:].reshape(32, G)
    packed = _sc_repack(embedding.T, tail_rm)
    table_rm = packed.reshape(V, D)
    out = _sc_gather(table_rm, idx_g)
    return (out.reshape(H, 8, G, 8, G)
            .transpose((2, 4, 0, 1, 3))
            .reshape(input_ids.shape[0], H, D))


# speedup vs baseline: 2.7634x; 1.4448x over previous
import functools

import jax
import jax.numpy as jnp
from jax import lax
from jax.experimental import pallas as pl
from jax.experimental.pallas import tpu as pltpu
from jax.experimental.pallas import tpu_sc as plsc

NC = 2
NS = 16
NW = NC * NS
G = 128
H = 50
D = 64
NB = 16384 // G * H
BPW = NB // NW
RING = 4


V = 1000000
NJB = V // G


def _sc_repack(table_t, tail_rm):
    mesh = plsc.VectorSubcoreMesh(core_axis_name="c", subcore_axis_name="s")

    @functools.partial(
        pl.kernel,
        out_type=jax.ShapeDtypeStruct((V // 2, G), jnp.float32),
        mesh=mesh,
        scratch_types=(
            [pltpu.VMEM((D, G), jnp.float32)] * 8
            + [pltpu.SemaphoreType.DMA] * 8
        ),
        compiler_params=pltpu.CompilerParams(needs_layout_passes=False),
    )
    def k(tab_hbm, tail_hbm, out_hbm, *bufs):
        sbuf = bufs[0:4]
        wbuf = bufs[4:8]
        isem = bufs[8:12]
        osem = bufs[12:16]

        wid = lax.axis_index("s") * NC + lax.axis_index("c")

        lanes = lax.iota(jnp.int32, 16)
        perm = [(lanes + q) & 15 for q in range(16)]

        def fire_load(kk, p):
            j = wid + kk * NW
            pltpu.async_copy(tab_hbm.at[:, pl.ds(j * G, G)], sbuf[p], isem[p])

        def wait_load(p):
            pltpu.make_async_copy(tab_hbm.at[:, pl.ds(0, G)],
                                  sbuf[p], isem[p]).wait()

        def transpose(p, rgroups):
            @plsc.parallel_loop(0, rgroups * 16, step=16, unroll=4)
            def _(r0):
                rows = lanes + r0
                rhalf = rows >> 1
                rodd = (rows & 1) << 6
                for cq in range(D // 16):
                    for q in range(16):
                        cvec = perm[q] + cq * 16
                        v = plsc.load_gather(sbuf[p], [cvec, rows])
                        plsc.store_scatter(wbuf[p], [rhalf, rodd + cvec], v)

        def fire_store(kk, p):
            j = wid + kk * NW
            pltpu.async_copy(wbuf[p], out_hbm.at[pl.ds(j * (G // 2), D)],
                             osem[p])

        def wait_store(p):
            pltpu.make_async_copy(wbuf[p], out_hbm.at[pl.ds(0, D)],
                                  osem[p]).wait()

        KMIN = NJB // NW
        for p in range(4):
            fire_load(p, p)

        def body(i, carry):
            for u in range(4):
                kk = 4 * i + u
                wait_load(u)

                @pl.when(i > 0)
                def _():
                    wait_store(u)

                transpose(u, 8)
                fire_store(kk, u)

                nxt = kk + 4

                @pl.when(nxt < KMIN)
                def _():
                    fire_load(nxt, u)

            return carry

        lax.fori_loop(0, KMIN // 4, body, None)

        @pl.when(wid < NJB - KMIN * NW)
        def _():
            fire_load(KMIN, 0)
            wait_load(0)
            wait_store(0)
            transpose(0, 8)
            fire_store(KMIN, 0)
            wait_store(0)

        wait_store(1)
        wait_store(2)
        wait_store(3)

        @pl.when(wid >= NJB - KMIN * NW)
        def _():
            wait_store(0)

        @pl.when(wid == NW - 1)
        def _():
            pltpu.sync_copy(tail_hbm, wbuf[0].at[pl.ds(0, 32)])
            pltpu.sync_copy(wbuf[0].at[pl.ds(0, 32)],
                            out_hbm.at[pl.ds(NJB * (G // 2), 32)])

    return k(table_t, tail_rm)


def _sc_gather(table, idx_g):
    mesh = plsc.VectorSubcoreMesh(core_axis_name="c", subcore_axis_name="s")

    @functools.partial(
        pl.kernel,
        out_type=jax.ShapeDtypeStruct((H, 8, G, 1024), jnp.float32),
        mesh=mesh,
        scratch_types=(
            [pltpu.VMEM((BPW, G), jnp.int32)]
            + [pltpu.VMEM((G, D), jnp.float32)] * RING
            + [pltpu.VMEM((8, G * 8), jnp.float32)] * RING
            + [pltpu.SemaphoreType.DMA] * (2 * RING)
        ),
        compiler_params=pltpu.CompilerParams(use_tc_tiling_on_sc=False,
                                             needs_layout_passes=False),
    )
    def k(table_hbm, idx_hbm, out_hbm, idx_v, *bufs):
        gbuf = bufs[:RING]
        tbuf = bufs[RING:2 * RING]
        gsem = bufs[2 * RING:3 * RING]
        ssem = bufs[3 * RING:4 * RING]

        wid = lax.axis_index("s") * NC + lax.axis_index("c")
        t_base = wid * BPW

        pltpu.sync_copy(idx_hbm.at[pl.ds(t_base, BPW)], idx_v)

        lanes = lax.iota(jnp.int32, 16)
        perm = [(lanes + q) & 15 for q in range(16)]
        perm_hi = [p >> 3 for p in perm]
        perm_lo = [(p & 7) * G for p in perm]

        def fire_gather(n, p):
            pltpu.async_copy(table_hbm.at[idx_v.at[n]], gbuf[p], gsem[p])

        def wait_gather(p):
            pltpu.make_async_copy(table_hbm.at[pl.ds(0, G)],
                                  gbuf[p], gsem[p]).wait()

        def transpose(p):
            @plsc.parallel_loop(0, G, step=16, unroll=4)
            def _(r0):
                rows = lanes + r0
                for cq in range(D // 16):
                    for q in range(16):
                        v = plsc.load_gather(gbuf[p], [rows, perm[q] + cq * 16])
                        plsc.store_scatter(
                            tbuf[p], [perm_hi[q] + cq * 2, perm_lo[q] + rows],
                            v)

        def fire_store(n, p):
            t = t_base + n
            h = t >> 7
            j = t & (G - 1)
            pltpu.async_copy(tbuf[p], out_hbm.at[h, pl.ds(0, 8), j], ssem[p])

        def wait_store(p):
            pltpu.make_async_copy(tbuf[p], out_hbm.at[0, pl.ds(0, 8), 0],
                                  ssem[p]).wait()

        for p in range(RING):
            fire_gather(p, p)

        def body(i, carry):
            for u in range(RING):
                n = RING * i + u
                wait_gather(u)

                @pl.when(i > 0)
                def _():
                    wait_store(u)

                transpose(u)
                fire_store(n, u)

                @pl.when(i < BPW // RING - 1)
                def _():
                    fire_gather(n + RING, u)

            return carry

        lax.fori_loop(0, BPW // RING, body, None)
        for p in range(RING):
            wait_store(p)

    return k(table, idx_g)


def kernel(input_ids, embedding):
    idx_g = input_ids.astype(jnp.int32).T.reshape(NB, G)
    tail_rm = embedding[V - D:].reshape(32, G)
    packed = _sc_repack(embedding.T, tail_rm)
    table_rm = packed.reshape(V, D)
    out = _sc_gather(table_rm, idx_g)
    return (out.reshape(H, 8, G, 8, G)
            .transpose((2, 4, 0, 1, 3))
            .reshape(input_ids.shape[0], H, D))
